# Initial kernel scaffold; baseline (speedup 1.0000x reference)
import jax
import jax.numpy as jnp
from jax.experimental import pallas as pl


def kernel(input_ids, word_table, token_type_table):
    raise NotImplementedError("write your pallas kernel here")


# speedup vs baseline: 1.0131x; 1.0131x over previous
import functools

import jax
import jax.numpy as jnp
from jax import lax
from jax.experimental import pallas as pl
from jax.experimental.pallas import tpu as pltpu
from jax.experimental.pallas import tpu_sc as plsc

VOCAB = 250002
DIM = 1024
B = 2
S = 4096

NC = 2
NS = 16
NW = NC * NS
N = B * S
PER_W = N // NW
CHUNK = 32
NCHUNK = PER_W // CHUNK
LANES = 16
NCOL = DIM // LANES

_mesh = plsc.VectorSubcoreMesh(core_axis_name="c", subcore_axis_name="s")


@functools.partial(
    pl.kernel,
    mesh=_mesh,
    out_type=jax.ShapeDtypeStruct((N, DIM), jnp.float32),
    scratch_types=[
        pltpu.VMEM((PER_W,), jnp.int32),
        pltpu.VMEM((DIM,), jnp.float32),
        pltpu.VMEM((CHUNK, DIM), jnp.float32),
        pltpu.SemaphoreType.DMA,
    ],
)
def _embed(ids_hbm, tt_hbm, table_hbm, out_hbm, idx_v, tt_v, rows_v, sem):
    wid = lax.axis_index("s") * NC + lax.axis_index("c")
    base = wid * PER_W
    pltpu.sync_copy(ids_hbm.at[pl.ds(base, PER_W)], idx_v)
    pltpu.sync_copy(tt_hbm, tt_v)
    for c in range(NCHUNK):
        pltpu.async_copy(
            table_hbm.at[idx_v.at[pl.ds(c * CHUNK, CHUNK)]], rows_v, sem
        ).wait()

        def col(j, carry):
            ttv = tt_v[pl.ds(j * LANES, LANES)]
            for i in range(CHUNK):
                rows_v[i, pl.ds(j * LANES, LANES)] += ttv
            return carry

        lax.fori_loop(0, NCOL, col, 0)
        pltpu.sync_copy(rows_v, out_hbm.at[pl.ds(base + c * CHUNK, CHUNK)])


def kernel(input_ids, word_table, token_type_table):
    ids = input_ids.reshape(-1).astype(jnp.int32)
    tt = token_type_table.reshape(-1)
    out = _embed(ids, tt, word_table)
    return out.reshape(B, S, DIM)


# speedup vs baseline: 1.3487x; 1.3313x over previous
import functools

import jax
import jax.numpy as jnp
from jax import lax
from jax.experimental import pallas as pl
from jax.experimental.pallas import tpu as pltpu
from jax.experimental.pallas import tpu_sc as plsc

VOCAB = 250002
DIM = 1024
B = 2
S = 4096

NC = 2
NS = 16
NW = NC * NS
N = B * S
PER_W = N // NW
CHUNK = 32
NCHUNK = PER_W // CHUNK
NBUF = 3
LANES = 16
NCOL = DIM // LANES

_mesh = plsc.VectorSubcoreMesh(core_axis_name="c", subcore_axis_name="s")


@functools.partial(
    pl.kernel,
    mesh=_mesh,
    out_type=jax.ShapeDtypeStruct((N, DIM), jnp.float32),
    scratch_types=[
        pltpu.VMEM((PER_W,), jnp.int32),
        pltpu.VMEM((DIM,), jnp.float32),
        pltpu.VMEM((NBUF, CHUNK, DIM), jnp.float32),
        pltpu.SemaphoreType.DMA((NBUF,)),
        pltpu.SemaphoreType.DMA((NBUF,)),
    ],
)
def _embed(ids_hbm, tt_hbm, table_hbm, out_hbm, idx_v, tt_v, bufs, gsem, osem):
    wid = lax.axis_index("s") * NC + lax.axis_index("c")
    base = wid * PER_W
    pltpu.sync_copy(ids_hbm.at[pl.ds(base, PER_W)], idx_v)
    pltpu.sync_copy(tt_hbm, tt_v)

    def gather(c):
        b = c % NBUF
        return pltpu.async_copy(
            table_hbm.at[idx_v.at[pl.ds(c * CHUNK, CHUNK)]], bufs.at[b], gsem.at[b]
        )

    def scatter(c):
        b = c % NBUF
        return pltpu.async_copy(
            bufs.at[b], out_hbm.at[pl.ds(base + c * CHUNK, CHUNK)], osem.at[b]
        )

    def add_tt(c):
        b = c % NBUF

        def col(j, carry):
            ttv = tt_v[pl.ds(j * LANES, LANES)]
            for i in range(CHUNK):
                bufs[b, i, pl.ds(j * LANES, LANES)] += ttv
            return carry

        lax.fori_loop(0, NCOL, col, 0)

    gathers = [None] * NCHUNK
    scatters = [None] * NCHUNK
    for c in range(NBUF - 1):
        gathers[c] = gather(c)
    for c in range(NBUF - 1, NCHUNK + NBUF - 1):
        if c < NCHUNK:
            if c >= NBUF:
                scatters[c - NBUF].wait()
            gathers[c] = gather(c)
        p = c - (NBUF - 1)
        gathers[p].wait()
        add_tt(p)
        scatters[p] = scatter(p)
    for p in range(NCHUNK - NBUF, NCHUNK):
        if p >= 0:
            scatters[p].wait()


def kernel(input_ids, word_table, token_type_table):
    ids = input_ids.reshape(-1).astype(jnp.int32)
    tt = token_type_table.reshape(-1)
    out = _embed(ids, tt, word_table)
    return out.reshape(B, S, DIM)
